# baseline (speedup 1.0000x reference)
import jax
import jax.numpy as jnp
from jax import lax
from jax.experimental import pallas as pl
from jax.experimental.pallas import tpu as pltpu
from jax.experimental.pallas import tpu_sc as plsc

D = 64
B, L = 64, 2048
N = B * L
N_T, N_B, N_P, N_K, N_D = 64, 128, 128, 256, 128

NC, NS = 2, 16
NW = NC * NS
TPW = N // NW
IG = 128
CHUNK = 256
NG = CHUNK // IG
NCHUNK = TPW // CHUNK
CPB = L // CHUNK


def _project_body(tt, bt, pt, kt, dt, w, b, otb, opd, otok):
    bias = b[0, :]
    p_t = jnp.dot(tt[...], w[0:64, :], preferred_element_type=jnp.float32) + bias
    p_b = jnp.dot(bt[...], w[64:128, :], preferred_element_type=jnp.float32)
    p_p = jnp.dot(pt[...], w[128:192, :], preferred_element_type=jnp.float32)
    p_k = jnp.dot(kt[...], w[192:256, :], preferred_element_type=jnp.float32)
    p_d = jnp.dot(dt[...], w[256:320, :], preferred_element_type=jnp.float32)
    for i in range(N_T):
        otb[pl.ds(i * N_B, N_B), :] = p_b + p_t[i:i + 1, :]
    for i in range(N_P):
        opd[pl.ds(i * N_D, N_D), :] = p_d + p_p[i:i + 1, :]
    otok[...] = p_k


def _project(tt, bt, pt, kt, dt, w, b):
    return pl.pallas_call(
        _project_body,
        out_shape=[
            jax.ShapeDtypeStruct((N_T * N_B, D), jnp.float32),
            jax.ShapeDtypeStruct((N_P * N_D, D), jnp.float32),
            jax.ShapeDtypeStruct((N_K, D), jnp.float32),
        ],
    )(tt, bt, pt, kt, dt, w, b.reshape(1, D))


def _lookup_body(ttb, tpd, ttok, i0, i1, i2, i3, i4, out,
                 vi0, vi1, ci0, ci1, ra0, ra1, rb0, rb1, rc0, rc1,
                 sidx, sg, so):
    cid = lax.axis_index("c")
    sid = lax.axis_index("s")
    wid = sid * NC + cid
    row0 = wid * (TPW // IG)
    vi = (vi0, vi1)
    ci = (ci0, ci1)
    ra = (ra0, ra1)
    rb = (rb0, rb1)
    rc = (rc0, rc1)
    idx_arrs = (i0, i1, i2, i3, i4)

    def idx_cps(k):
        r = pl.ds(row0 + k * NG, NG)
        return [pltpu.make_async_copy(idx_arrs[f].at[r], vi[k & 1].at[f], sidx)
                for f in range(5)]

    def gather_cps(k):
        p = k & 1
        cps = []
        for g in range(NG):
            dst = pl.ds(g * IG, IG)
            cps.append(pltpu.make_async_copy(
                ttb.at[ci[p].at[g, 0]], ra[p].at[dst], sg))
            cps.append(pltpu.make_async_copy(
                tpd.at[ci[p].at[g, 1]], rb[p].at[dst], sg))
            cps.append(pltpu.make_async_copy(
                ttok.at[vi[p].at[3, g]], rc[p].at[dst], sg))
        return cps

    def out_cp(k):
        bi = 2 * wid + (k // CPB)
        l0 = (k % CPB) * CHUNK
        return pltpu.make_async_copy(
            ra[k & 1], out.at[bi, pl.ds(l0, CHUNK)], so)

    def combine(k):
        p = k & 1
        vip, cip = vi[p], ci[p]

        def cb(j, c):
            g = j // (IG // 16)
            col = (j % (IG // 16)) * 16
            sl = pl.ds(col, 16)
            cip[g, 0, sl] = vip[0, g, sl] * N_B + vip[1, g, sl]
            cip[g, 1, sl] = vip[2, g, sl] * N_D + vip[4, g, sl]
            return c
        lax.fori_loop(0, NG * (IG // 16), cb, 0)

    def accum(k):
        p = k & 1
        rap, rbp, rcp = ra[p], rb[p], rc[p]

        def ab(t, c):
            for cc in range(D // 16):
                sl = pl.ds(cc * 16, 16)
                rap[t, sl] = rap[t, sl] + rbp[t, sl] + rcp[t, sl]
            return c
        lax.fori_loop(0, CHUNK, ab, 0)

    for cp in idx_cps(0):
        cp.start()
    for cp in idx_cps(0):
        cp.wait()
    combine(0)
    for cp in gather_cps(0):
        cp.start()
    if NCHUNK > 1:
        for cp in idx_cps(1):
            cp.start()

    for k in range(NCHUNK):
        if k + 1 < NCHUNK:
            for cp in idx_cps(k + 1):
                cp.wait()
            combine(k + 1)
        for cp in gather_cps(k):
            cp.wait()
        if k >= 1:
            out_cp(k - 1).wait()
        if k + 1 < NCHUNK:
            for cp in gather_cps(k + 1):
                cp.start()
            if k + 2 < NCHUNK:
                for cp in idx_cps(k + 2):
                    cp.start()
        accum(k)
        out_cp(k).start()
    out_cp(NCHUNK - 1).wait()


def _lookup(ttb, tpd, ttok, i0, i1, i2, i3, i4):
    mesh = plsc.VectorSubcoreMesh(core_axis_name="c", subcore_axis_name="s")
    f = pl.kernel(
        _lookup_body,
        out_type=jax.ShapeDtypeStruct((B, L, D), jnp.float32),
        mesh=mesh,
        scratch_types=[
            pltpu.VMEM((5, NG, IG), jnp.int32),
            pltpu.VMEM((5, NG, IG), jnp.int32),
            pltpu.VMEM((NG, 2, IG), jnp.int32),
            pltpu.VMEM((NG, 2, IG), jnp.int32),
            pltpu.VMEM((CHUNK, D), jnp.float32),
            pltpu.VMEM((CHUNK, D), jnp.float32),
            pltpu.VMEM((CHUNK, D), jnp.float32),
            pltpu.VMEM((CHUNK, D), jnp.float32),
            pltpu.VMEM((CHUNK, D), jnp.float32),
            pltpu.VMEM((CHUNK, D), jnp.float32),
            pltpu.SemaphoreType.DMA,
            pltpu.SemaphoreType.DMA,
            pltpu.SemaphoreType.DMA,
        ],
        compiler_params=pltpu.CompilerParams(use_tc_tiling_on_sc=False),
    )
    return f(ttb, tpd, ttok, i0, i1, i2, i3, i4)


def kernel(Tempo, Bar, Position, Token, Duration, tempo_table, bar_table,
           pos_table, token_table, dur_table, W_dec, b_dec):
    ttb, tpd, ttok = _project(tempo_table, bar_table, pos_table,
                              token_table, dur_table, W_dec, b_dec)
    shp = (N // IG, IG)
    return _lookup(
        ttb, tpd, ttok,
        Tempo.reshape(shp), Bar.reshape(shp), Position.reshape(shp),
        Token.reshape(shp), Duration.reshape(shp),
    )
